# baseline (speedup 1.0000x reference)
import jax
import jax.numpy as jnp
from jax import lax
from jax.experimental import pallas as pl
from jax.experimental.pallas import tpu as pltpu
from jax.experimental.pallas import tpu_sc as plsc

_EOS = 3
_NEG_INF = float("-inf")
_IMAX = 2**31 - 1

_B, _K, _V = 64, 5, 100000
_NT = 32
_STRIPE = 3200
_STRIPE_L = 768
_COV = 31 * _STRIPE + _STRIPE_L
_TAILW = _V - _COV
_NV, _NV_L = _STRIPE // 16, _STRIPE_L // 16
_CW = 80
_U = 8


def _sc_scan(probs_hbm, sp_hbm, len_hbm, cvi_hbm,
             sp_v, len_v, buf, cv_s, sem):
    c = lax.axis_index("c")
    s = lax.axis_index("s")
    wid = s * 2 + c
    pltpu.sync_copy(sp_hbm, sp_v)
    pltpu.sync_copy(len_hbm, len_v)
    count = sp_v[pl.ds(0, 16)][0]
    base_col = wid * _STRIPE
    lane = lax.iota(jnp.int32, 16)
    neg = jnp.full((16,), _NEG_INF, jnp.float32)
    zero = jnp.zeros((16,), jnp.int32)

    def make_parts(nv, stripe_len):
        unroll = _U

        def start(t, slot):
            b = sp_v[pl.ds(1 + t, 16)][0]
            pltpu.make_async_copy(
                probs_hbm.at[b, :, pl.ds(base_col, stripe_len)],
                buf.at[slot, :, pl.ds(0, stripe_len)],
                sem.at[slot]).start()

        def task_body(t, carry_dummy):
            slot = lax.rem(t, 2)
            b = sp_v[pl.ds(1 + t, 16)][0]

            @pl.when(t + 1 < count)
            def _():
                start(t + 1, 1 - slot)

            pltpu.make_async_copy(
                probs_hbm.at[b, :, pl.ds(base_col, stripe_len)],
                buf.at[slot, :, pl.ds(0, stripe_len)],
                sem.at[slot]).wait()

            for kk in range(_K):
                rlen = len_v[pl.ds(b * _K + kk, 16)][0]

                @pl.when(rlen == 0)
                def _(kk=kk):
                    def scan_body(v, carry, kk=kk):
                        rv = list(carry[:5])
                        ri = list(carry[5:])
                        for u in range(unroll):
                            x = buf[slot, kk, pl.ds((v * unroll + u) * 16, 16)]
                            ix = lane + (base_col + (v * unroll + u) * 16)
                            for lvl in range(5):
                                g = x > rv[lvl]
                                rv[lvl], x = (jnp.where(g, x, rv[lvl]),
                                              jnp.where(g, rv[lvl], x))
                                ri[lvl], ix = (jnp.where(g, ix, ri[lvl]),
                                               jnp.where(g, ri[lvl], ix))
                        return tuple(rv) + tuple(ri)

                    init = (neg,) * 5 + (zero,) * 5
                    fin = lax.fori_loop(0, nv // unroll, scan_body, init)
                    for lvl in range(5):
                        cv_s[pl.ds(lvl * 16, 16)] = fin[lvl]
                        cv_s[pl.ds(_CW + lvl * 16, 16)] = (
                            fin[5 + lvl].astype(jnp.float32))
                    out_base = ((b * _K + kk) * _NT + wid) * 2 * _CW
                    pltpu.sync_copy(cv_s, cvi_hbm.at[pl.ds(out_base, 2 * _CW)])
            return carry_dummy
        return start, task_body

    @pl.when(jnp.logical_and(wid < _NT - 1, count > 0))
    def _():
        start, body = make_parts(_NV, _STRIPE)
        start(0, 0)
        lax.fori_loop(0, count, body, 0)

    @pl.when(jnp.logical_and(wid == _NT - 1, count > 0))
    def _():
        start, body = make_parts(_NV_L, _STRIPE_L)
        start(0, 0)
        lax.fori_loop(0, count, body, 0)


def _merge_kernel(cv_ref, ci_ref, tail_ref, lp_ref, pen_ref, len_ref,
                  sc_ref, pv_ref, ix_ref):
    cv = cv_ref[...]
    ci = ci_ref[...]
    tail = tail_ref[...]
    lp = lp_ref[...]
    pen = pen_ref[...]
    done = len_ref[...] != 0
    B, K, W = cv.shape

    krow = lax.broadcasted_iota(jnp.int32, (B, K, W), 1)
    x_act = jnp.where(done, _NEG_INF, (lp + cv) / pen)
    i_act = jnp.where(done, _IMAX, krow * _V + ci)
    p_act = jnp.where(done, _NEG_INF, lp + cv)

    krowt = lax.broadcasted_iota(jnp.int32, (B, K, _TAILW), 1)
    colt = lax.broadcasted_iota(jnp.int32, (B, K, _TAILW), 2)
    x_tail = jnp.where(done, _NEG_INF, (lp + tail) / pen)
    i_tail = jnp.where(done, _IMAX, krowt * _V + _COV + colt)
    p_tail = jnp.where(done, _NEG_INF, lp + tail)

    krow1 = lax.broadcasted_iota(jnp.int32, (B, K, 1), 1)
    x_eos = jnp.where(done, lp / pen, _NEG_INF)
    i_eos = jnp.where(done, krow1 * _V + _EOS, _IMAX)
    p_eos = jnp.where(done, lp, _NEG_INF)

    X = jnp.concatenate([x_act, x_tail, x_eos], axis=2)
    I = jnp.concatenate([i_act, i_tail, i_eos], axis=2)
    P = jnp.concatenate([p_act, p_tail, p_eos], axis=2)

    scs, pvs, ixs = [], [], []
    for _ in range(5):
        m = jnp.max(jnp.max(X, axis=2), axis=1)
        mb = m[:, None, None]
        csel = jnp.min(jnp.min(jnp.where(X == mb, I, _IMAX), axis=2), axis=1)
        cb = csel[:, None, None]
        pv = jnp.max(jnp.max(jnp.where(I == cb, P, _NEG_INF), axis=2), axis=1)
        scs.append(m)
        pvs.append(pv)
        ixs.append(csel)
        X = jnp.where(I == cb, _NEG_INF, X)

    sc_ref[...] = jnp.stack(scs, axis=1)
    pv_ref[...] = jnp.stack(pvs, axis=1)
    ix_ref[...] = jnp.stack(ixs, axis=1)


def kernel(probs, log_probs, lengths, i, k):
    B, K, V = probs.shape

    batch_active = jnp.any(lengths == 0, axis=1)
    order = jnp.argsort(jnp.logical_not(batch_active),
                        stable=True).astype(jnp.int32)
    count = jnp.sum(batch_active).astype(jnp.int32)
    sp = jnp.concatenate([count[None], order,
                          jnp.zeros(23, jnp.int32)])
    len_pad = jnp.concatenate(
        [lengths.reshape(B * K), jnp.zeros(16, jnp.int32)])

    mesh = plsc.VectorSubcoreMesh(core_axis_name="c", subcore_axis_name="s")
    scan = pl.kernel(
        _sc_scan,
        out_type=jax.ShapeDtypeStruct((_B * _K * _NT * 2 * _CW,),
                                      jnp.float32),
        mesh=mesh,
        scratch_types=[
            pltpu.VMEM((88,), jnp.int32),
            pltpu.VMEM((336,), jnp.int32),
            pltpu.VMEM((2, _K, _STRIPE), jnp.float32),
            pltpu.VMEM((2 * _CW,), jnp.float32),
            pltpu.SemaphoreType.DMA((2,)),
        ],
    )
    cvi = scan(probs, sp, len_pad)
    co = cvi.reshape(_B, _K, _NT, 2 * _CW)
    cv = co[..., :_CW]
    ci = co[..., _CW:].astype(jnp.int32)

    eff = jnp.where(lengths == 0, i + 1, lengths).astype(jnp.float32)
    pen = jnp.power((5.0 + eff) / 6.0, 0.8)

    W = _NT * _CW
    tail = lax.slice(probs, (0, 0, _COV), (B, K, V))
    full = lambda shp: pl.BlockSpec(shp, lambda: (0,) * len(shp))
    sc, pv, ix = pl.pallas_call(
        _merge_kernel,
        in_specs=[
            full((B, K, W)), full((B, K, W)), full((B, K, _TAILW)),
            full((B, K, 1)), full((B, K, 1)), full((B, K, 1)),
        ],
        out_specs=(full((B, 5)), full((B, 5)), full((B, 5))),
        out_shape=(
            jax.ShapeDtypeStruct((B, 5), jnp.float32),
            jax.ShapeDtypeStruct((B, 5), jnp.float32),
            jax.ShapeDtypeStruct((B, 5), jnp.int32),
        ),
    )(cv.reshape(B, K, W), ci.reshape(B, K, W), tail,
      log_probs.reshape(B, K, 1), pen.reshape(B, K, 1),
      lengths.reshape(B, K, 1))

    best_idx = ix + jnp.asarray(k - K, jnp.int32)
    best_beams = best_idx // V
    best_tokens = best_idx % V
    return sc, pv, best_beams, best_tokens


# speedup vs baseline: 1.0021x; 1.0021x over previous
import jax
import jax.numpy as jnp
from jax import lax
from jax.experimental import pallas as pl
from jax.experimental.pallas import tpu as pltpu
from jax.experimental.pallas import tpu_sc as plsc

_EOS = 3
_NEG_INF = float("-inf")
_IMAX = 2**31 - 1

_B, _K, _V = 64, 5, 100000
_NT = 32
_STRIPE = 3200
_STRIPE_L = 768
_COV = 31 * _STRIPE + _STRIPE_L
_TAILW = _V - _COV
_NV, _NV_L = _STRIPE // 16, _STRIPE_L // 16
_CW = 80
_U = 4


def _sc_scan(probs_hbm, sp_hbm, len_hbm, cvi_hbm,
             sp_v, len_v, buf, cv_s, sem):
    c = lax.axis_index("c")
    s = lax.axis_index("s")
    wid = s * 2 + c
    pltpu.sync_copy(sp_hbm, sp_v)
    pltpu.sync_copy(len_hbm, len_v)
    count = sp_v[pl.ds(0, 16)][0]
    base_col = wid * _STRIPE
    lane = lax.iota(jnp.int32, 16)
    neg = jnp.full((16,), _NEG_INF, jnp.float32)
    zero = jnp.zeros((16,), jnp.int32)

    def make_parts(nv, stripe_len):
        unroll = _U

        def start(t, slot):
            b = sp_v[pl.ds(1 + t, 16)][0]
            pltpu.make_async_copy(
                probs_hbm.at[b, :, pl.ds(base_col, stripe_len)],
                buf.at[slot, :, pl.ds(0, stripe_len)],
                sem.at[slot]).start()

        def task_body(t, carry_dummy):
            slot = lax.rem(t, 2)
            b = sp_v[pl.ds(1 + t, 16)][0]

            @pl.when(t + 1 < count)
            def _():
                start(t + 1, 1 - slot)

            pltpu.make_async_copy(
                probs_hbm.at[b, :, pl.ds(base_col, stripe_len)],
                buf.at[slot, :, pl.ds(0, stripe_len)],
                sem.at[slot]).wait()

            for kk in range(_K):
                rlen = len_v[pl.ds(b * _K + kk, 16)][0]

                @pl.when(rlen == 0)
                def _(kk=kk):
                    def scan_body(v, carry, kk=kk):
                        rv = list(carry[:5])
                        ri = list(carry[5:])
                        for u in range(unroll):
                            x = buf[slot, kk, pl.ds((v * unroll + u) * 16, 16)]
                            ix = lane + (base_col + (v * unroll + u) * 16)
                            for lvl in range(5):
                                g = x > rv[lvl]
                                rv[lvl], x = (jnp.where(g, x, rv[lvl]),
                                              jnp.where(g, rv[lvl], x))
                                ri[lvl], ix = (jnp.where(g, ix, ri[lvl]),
                                               jnp.where(g, ri[lvl], ix))
                        return tuple(rv) + tuple(ri)

                    init = (neg,) * 5 + (zero,) * 5
                    fin = lax.fori_loop(0, nv // unroll, scan_body, init)
                    for lvl in range(5):
                        cv_s[pl.ds(lvl * 16, 16)] = fin[lvl]
                        cv_s[pl.ds(_CW + lvl * 16, 16)] = (
                            fin[5 + lvl].astype(jnp.float32))
                    out_base = ((b * _K + kk) * _NT + wid) * 2 * _CW
                    pltpu.sync_copy(cv_s, cvi_hbm.at[pl.ds(out_base, 2 * _CW)])
            return carry_dummy
        return start, task_body

    @pl.when(jnp.logical_and(wid < _NT - 1, count > 0))
    def _():
        start, body = make_parts(_NV, _STRIPE)
        start(0, 0)
        lax.fori_loop(0, count, body, 0)

    @pl.when(jnp.logical_and(wid == _NT - 1, count > 0))
    def _():
        start, body = make_parts(_NV_L, _STRIPE_L)
        start(0, 0)
        lax.fori_loop(0, count, body, 0)


def _merge_kernel(cv_ref, ci_ref, tail_ref, lp_ref, pen_ref, len_ref,
                  sc_ref, pv_ref, ix_ref):
    cv = cv_ref[...]
    ci = ci_ref[...]
    tail = tail_ref[...]
    lp = lp_ref[...]
    pen = pen_ref[...]
    done = len_ref[...] != 0
    B, K, W = cv.shape

    krow = lax.broadcasted_iota(jnp.int32, (B, K, W), 1)
    x_act = jnp.where(done, _NEG_INF, (lp + cv) / pen)
    i_act = jnp.where(done, _IMAX, krow * _V + ci)
    p_act = jnp.where(done, _NEG_INF, lp + cv)

    krowt = lax.broadcasted_iota(jnp.int32, (B, K, _TAILW), 1)
    colt = lax.broadcasted_iota(jnp.int32, (B, K, _TAILW), 2)
    x_tail = jnp.where(done, _NEG_INF, (lp + tail) / pen)
    i_tail = jnp.where(done, _IMAX, krowt * _V + _COV + colt)
    p_tail = jnp.where(done, _NEG_INF, lp + tail)

    krow1 = lax.broadcasted_iota(jnp.int32, (B, K, 1), 1)
    x_eos = jnp.where(done, lp / pen, _NEG_INF)
    i_eos = jnp.where(done, krow1 * _V + _EOS, _IMAX)
    p_eos = jnp.where(done, lp, _NEG_INF)

    X = jnp.concatenate([x_act, x_tail, x_eos], axis=2)
    I = jnp.concatenate([i_act, i_tail, i_eos], axis=2)
    P = jnp.concatenate([p_act, p_tail, p_eos], axis=2)

    scs, pvs, ixs = [], [], []
    for _ in range(5):
        m = jnp.max(jnp.max(X, axis=2), axis=1)
        mb = m[:, None, None]
        csel = jnp.min(jnp.min(jnp.where(X == mb, I, _IMAX), axis=2), axis=1)
        cb = csel[:, None, None]
        pv = jnp.max(jnp.max(jnp.where(I == cb, P, _NEG_INF), axis=2), axis=1)
        scs.append(m)
        pvs.append(pv)
        ixs.append(csel)
        X = jnp.where(I == cb, _NEG_INF, X)

    sc_ref[...] = jnp.stack(scs, axis=1)
    pv_ref[...] = jnp.stack(pvs, axis=1)
    ix_ref[...] = jnp.stack(ixs, axis=1)


def kernel(probs, log_probs, lengths, i, k):
    B, K, V = probs.shape

    batch_active = jnp.any(lengths == 0, axis=1)
    order = jnp.argsort(jnp.logical_not(batch_active),
                        stable=True).astype(jnp.int32)
    count = jnp.sum(batch_active).astype(jnp.int32)
    sp = jnp.concatenate([count[None], order,
                          jnp.zeros(23, jnp.int32)])
    len_pad = jnp.concatenate(
        [lengths.reshape(B * K), jnp.zeros(16, jnp.int32)])

    mesh = plsc.VectorSubcoreMesh(core_axis_name="c", subcore_axis_name="s")
    scan = pl.kernel(
        _sc_scan,
        out_type=jax.ShapeDtypeStruct((_B * _K * _NT * 2 * _CW,),
                                      jnp.float32),
        mesh=mesh,
        scratch_types=[
            pltpu.VMEM((88,), jnp.int32),
            pltpu.VMEM((336,), jnp.int32),
            pltpu.VMEM((2, _K, _STRIPE), jnp.float32),
            pltpu.VMEM((2 * _CW,), jnp.float32),
            pltpu.SemaphoreType.DMA((2,)),
        ],
    )
    cvi = scan(probs, sp, len_pad)
    co = cvi.reshape(_B, _K, _NT, 2 * _CW)
    cv = co[..., :_CW]
    ci = co[..., _CW:].astype(jnp.int32)

    eff = jnp.where(lengths == 0, i + 1, lengths).astype(jnp.float32)
    pen = jnp.power((5.0 + eff) / 6.0, 0.8)

    W = _NT * _CW
    tail = lax.slice(probs, (0, 0, _COV), (B, K, V))
    full = lambda shp: pl.BlockSpec(shp, lambda: (0,) * len(shp))
    sc, pv, ix = pl.pallas_call(
        _merge_kernel,
        in_specs=[
            full((B, K, W)), full((B, K, W)), full((B, K, _TAILW)),
            full((B, K, 1)), full((B, K, 1)), full((B, K, 1)),
        ],
        out_specs=(full((B, 5)), full((B, 5)), full((B, 5))),
        out_shape=(
            jax.ShapeDtypeStruct((B, 5), jnp.float32),
            jax.ShapeDtypeStruct((B, 5), jnp.float32),
            jax.ShapeDtypeStruct((B, 5), jnp.int32),
        ),
    )(cv.reshape(B, K, W), ci.reshape(B, K, W), tail,
      log_probs.reshape(B, K, 1), pen.reshape(B, K, 1),
      lengths.reshape(B, K, 1))

    best_idx = ix + jnp.asarray(k - K, jnp.int32)
    best_beams = best_idx // V
    best_tokens = best_idx % V
    return sc, pv, best_beams, best_tokens


# speedup vs baseline: 1.1074x; 1.1050x over previous
import jax
import jax.numpy as jnp
from jax import lax
from jax.experimental import pallas as pl
from jax.experimental.pallas import tpu as pltpu
from jax.experimental.pallas import tpu_sc as plsc

_EOS = 3
_NEG_INF = float("-inf")
_IMAX = 2**31 - 1

_B, _K, _V = 64, 5, 100000
_NT = 32
_STRIPE = 3200
_STRIPE_L = 768
_COV = 31 * _STRIPE + _STRIPE_L
_TAILW = _V - _COV
_NV, _NV_L = _STRIPE // 16, _STRIPE_L // 16
_CW = 80
_U = 4


def _sc_scan(probs_hbm, sp_hbm, len_hbm, cv_hbm, ci_hbm,
             sp_v, len_v, buf, cv_s, ci_s, sem):
    c = lax.axis_index("c")
    s = lax.axis_index("s")
    wid = s * 2 + c
    pltpu.sync_copy(sp_hbm, sp_v)
    pltpu.sync_copy(len_hbm, len_v)
    count = sp_v[pl.ds(0, 16)][0]
    base_col = wid * _STRIPE
    lane = lax.iota(jnp.int32, 16)
    neg = jnp.full((16,), _NEG_INF, jnp.float32)
    zero = jnp.zeros((16,), jnp.int32)

    def make_parts(nv, stripe_len):
        unroll = _U

        def start(t, slot):
            b = sp_v[pl.ds(1 + t, 16)][0]
            pltpu.make_async_copy(
                probs_hbm.at[b, :, pl.ds(base_col, stripe_len)],
                buf.at[slot, :, pl.ds(0, stripe_len)],
                sem.at[slot]).start()

        def task_body(t, carry_dummy):
            slot = lax.rem(t, 2)
            b = sp_v[pl.ds(1 + t, 16)][0]

            @pl.when(t + 1 < count)
            def _():
                start(t + 1, 1 - slot)

            pltpu.make_async_copy(
                probs_hbm.at[b, :, pl.ds(base_col, stripe_len)],
                buf.at[slot, :, pl.ds(0, stripe_len)],
                sem.at[slot]).wait()

            for kk in range(_K):
                rlen = len_v[pl.ds(b * _K + kk, 16)][0]

                @pl.when(rlen == 0)
                def _(kk=kk):
                    def scan_body(v, carry, kk=kk):
                        rv = list(carry[:5])
                        ri = list(carry[5:])
                        for u in range(unroll):
                            x = buf[slot, kk, pl.ds((v * unroll + u) * 16, 16)]
                            ix = lane + (base_col + (v * unroll + u) * 16)
                            for lvl in range(5):
                                g = x > rv[lvl]
                                rv[lvl], x = (jnp.where(g, x, rv[lvl]),
                                              jnp.where(g, rv[lvl], x))
                                ri[lvl], ix = (jnp.where(g, ix, ri[lvl]),
                                               jnp.where(g, ri[lvl], ix))
                        return tuple(rv) + tuple(ri)

                    init = (neg,) * 5 + (zero,) * 5
                    fin = lax.fori_loop(0, nv // unroll, scan_body, init)
                    for lvl in range(5):
                        cv_s[pl.ds(lvl * 16, 16)] = fin[lvl]
                        ci_s[pl.ds(lvl * 16, 16)] = fin[5 + lvl]
                    out_base = ((b * _K + kk) * _NT + wid) * _CW
                    pltpu.sync_copy(cv_s, cv_hbm.at[pl.ds(out_base, _CW)])
                    pltpu.sync_copy(ci_s, ci_hbm.at[pl.ds(out_base, _CW)])
            return carry_dummy
        return start, task_body

    @pl.when(jnp.logical_and(wid < _NT - 1, count > 0))
    def _():
        start, body = make_parts(_NV, _STRIPE)
        start(0, 0)
        lax.fori_loop(0, count, body, 0)

    @pl.when(jnp.logical_and(wid == _NT - 1, count > 0))
    def _():
        start, body = make_parts(_NV_L, _STRIPE_L)
        start(0, 0)
        lax.fori_loop(0, count, body, 0)


def _merge_kernel(cv_ref, ci_ref, tail_ref, lp_ref, pen_ref, len_ref,
                  sc_ref, pv_ref, ix_ref):
    cv = cv_ref[...]
    ci = ci_ref[...]
    tail = tail_ref[...]
    lp = lp_ref[...]
    pen = pen_ref[...]
    done = len_ref[...] != 0
    B, K, W = cv.shape

    krow = lax.broadcasted_iota(jnp.int32, (B, K, W), 1)
    x_act = jnp.where(done, _NEG_INF, (lp + cv) / pen)
    i_act = jnp.where(done, _IMAX, krow * _V + ci)
    p_act = jnp.where(done, _NEG_INF, lp + cv)

    krowt = lax.broadcasted_iota(jnp.int32, (B, K, _TAILW), 1)
    colt = lax.broadcasted_iota(jnp.int32, (B, K, _TAILW), 2)
    x_tail = jnp.where(done, _NEG_INF, (lp + tail) / pen)
    i_tail = jnp.where(done, _IMAX, krowt * _V + _COV + colt)
    p_tail = jnp.where(done, _NEG_INF, lp + tail)

    krow1 = lax.broadcasted_iota(jnp.int32, (B, K, 1), 1)
    x_eos = jnp.where(done, lp / pen, _NEG_INF)
    i_eos = jnp.where(done, krow1 * _V + _EOS, _IMAX)
    p_eos = jnp.where(done, lp, _NEG_INF)

    X = jnp.concatenate([x_act, x_tail, x_eos], axis=2)
    I = jnp.concatenate([i_act, i_tail, i_eos], axis=2)
    P = jnp.concatenate([p_act, p_tail, p_eos], axis=2)

    scs, pvs, ixs = [], [], []
    for _ in range(5):
        m = jnp.max(jnp.max(X, axis=2), axis=1)
        mb = m[:, None, None]
        csel = jnp.min(jnp.min(jnp.where(X == mb, I, _IMAX), axis=2), axis=1)
        cb = csel[:, None, None]
        pv = jnp.max(jnp.max(jnp.where(I == cb, P, _NEG_INF), axis=2), axis=1)
        scs.append(m)
        pvs.append(pv)
        ixs.append(csel)
        X = jnp.where(I == cb, _NEG_INF, X)

    sc_ref[...] = jnp.stack(scs, axis=1)
    pv_ref[...] = jnp.stack(pvs, axis=1)
    ix_ref[...] = jnp.stack(ixs, axis=1)


def kernel(probs, log_probs, lengths, i, k):
    B, K, V = probs.shape

    batch_active = jnp.any(lengths == 0, axis=1)
    order = jnp.argsort(jnp.logical_not(batch_active),
                        stable=True).astype(jnp.int32)
    count = jnp.sum(batch_active).astype(jnp.int32)
    sp = jnp.concatenate([count[None], order,
                          jnp.zeros(23, jnp.int32)])
    len_pad = jnp.concatenate(
        [lengths.reshape(B * K), jnp.zeros(16, jnp.int32)])

    mesh = plsc.VectorSubcoreMesh(core_axis_name="c", subcore_axis_name="s")
    scan = pl.kernel(
        _sc_scan,
        out_type=(
            jax.ShapeDtypeStruct((_B * _K * _NT * _CW,), jnp.float32),
            jax.ShapeDtypeStruct((_B * _K * _NT * _CW,), jnp.int32),
        ),
        mesh=mesh,
        scratch_types=[
            pltpu.VMEM((88,), jnp.int32),
            pltpu.VMEM((336,), jnp.int32),
            pltpu.VMEM((2, _K, _STRIPE), jnp.float32),
            pltpu.VMEM((_CW,), jnp.float32),
            pltpu.VMEM((_CW,), jnp.int32),
            pltpu.SemaphoreType.DMA((2,)),
        ],
    )
    cv, ci = scan(probs, sp, len_pad)

    eff = jnp.where(lengths == 0, i + 1, lengths).astype(jnp.float32)
    pen = jnp.power((5.0 + eff) / 6.0, 0.8)

    W = _NT * _CW
    tail = lax.slice(probs, (0, 0, _COV), (B, K, V))
    full = lambda shp: pl.BlockSpec(shp, lambda: (0,) * len(shp))
    sc, pv, ix = pl.pallas_call(
        _merge_kernel,
        in_specs=[
            full((B, K, W)), full((B, K, W)), full((B, K, _TAILW)),
            full((B, K, 1)), full((B, K, 1)), full((B, K, 1)),
        ],
        out_specs=(full((B, 5)), full((B, 5)), full((B, 5))),
        out_shape=(
            jax.ShapeDtypeStruct((B, 5), jnp.float32),
            jax.ShapeDtypeStruct((B, 5), jnp.float32),
            jax.ShapeDtypeStruct((B, 5), jnp.int32),
        ),
    )(cv.reshape(B, K, W), ci.reshape(B, K, W), tail,
      log_probs.reshape(B, K, 1), pen.reshape(B, K, 1),
      lengths.reshape(B, K, 1))

    best_idx = ix + jnp.asarray(k - K, jnp.int32)
    best_beams = best_idx // V
    best_tokens = best_idx % V
    return sc, pv, best_beams, best_tokens


# speedup vs baseline: 1.1308x; 1.0212x over previous
import jax
import jax.numpy as jnp
from jax import lax
from jax.experimental import pallas as pl
from jax.experimental.pallas import tpu as pltpu
from jax.experimental.pallas import tpu_sc as plsc

_EOS = 3
_NEG_INF = float("-inf")
_IMAX = 2**31 - 1

_B, _K, _V = 64, 5, 100000
_NT = 32
_STRIPE = 3200
_STRIPE_L = 768
_COV = 31 * _STRIPE + _STRIPE_L
_TAILW = _V - _COV
_NV, _NV_L = _STRIPE // 16, _STRIPE_L // 16
_CW = 80
_U = 4


def _sc_scan(probs_hbm, len_hbm, cv_hbm, ci_hbm,
             len_v, list_s, buf, cv_s, ci_s, sem):
    c = lax.axis_index("c")
    s = lax.axis_index("s")
    wid = s * 2 + c
    pltpu.sync_copy(len_hbm, len_v)
    base_col = wid * _STRIPE
    lane = lax.iota(jnp.int32, 16)
    neg = jnp.full((16,), _NEG_INF, jnp.float32)
    zero = jnp.zeros((16,), jnp.int32)

    def build(b, cnt):
        rl = [len_v[pl.ds(b * _K + kk, 16)][0] for kk in range(_K)]
        m = jnp.minimum(jnp.minimum(jnp.minimum(rl[0], rl[1]),
                                    jnp.minimum(rl[2], rl[3])), rl[4])
        act = m == 0

        @pl.when(act)
        def _():
            list_s[cnt] = b

        return cnt + jnp.where(act, 1, 0)

    count = lax.fori_loop(0, _B, build, 0)

    def make_parts(nv, stripe_len):
        unroll = _U

        def start(t, slot):
            b = list_s[t]
            pltpu.make_async_copy(
                probs_hbm.at[b, :, pl.ds(base_col, stripe_len)],
                buf.at[slot, :, pl.ds(0, stripe_len)],
                sem.at[slot]).start()

        def task_body(t, carry_dummy):
            slot = lax.rem(t, 2)
            b = list_s[t]

            @pl.when(t + 1 < count)
            def _():
                start(t + 1, 1 - slot)

            pltpu.make_async_copy(
                probs_hbm.at[b, :, pl.ds(base_col, stripe_len)],
                buf.at[slot, :, pl.ds(0, stripe_len)],
                sem.at[slot]).wait()

            for kk in range(_K):
                rlen = len_v[pl.ds(b * _K + kk, 16)][0]

                @pl.when(rlen == 0)
                def _(kk=kk):
                    def scan_body(v, carry, kk=kk):
                        rv = list(carry[:5])
                        ri = list(carry[5:])
                        for u in range(unroll):
                            x = buf[slot, kk, pl.ds((v * unroll + u) * 16, 16)]
                            ix = lane + (base_col + (v * unroll + u) * 16)
                            for lvl in range(5):
                                g = x > rv[lvl]
                                rv[lvl], x = (jnp.where(g, x, rv[lvl]),
                                              jnp.where(g, rv[lvl], x))
                                ri[lvl], ix = (jnp.where(g, ix, ri[lvl]),
                                               jnp.where(g, ri[lvl], ix))
                        return tuple(rv) + tuple(ri)

                    init = (neg,) * 5 + (zero,) * 5
                    fin = lax.fori_loop(0, nv // unroll, scan_body, init)
                    for lvl in range(5):
                        cv_s[pl.ds(lvl * 16, 16)] = fin[lvl]
                        ci_s[pl.ds(lvl * 16, 16)] = fin[5 + lvl]
                    out_base = ((b * _K + kk) * _NT + wid) * _CW
                    pltpu.sync_copy(cv_s, cv_hbm.at[pl.ds(out_base, _CW)])
                    pltpu.sync_copy(ci_s, ci_hbm.at[pl.ds(out_base, _CW)])
            return carry_dummy
        return start, task_body

    @pl.when(jnp.logical_and(wid < _NT - 1, count > 0))
    def _():
        start, body = make_parts(_NV, _STRIPE)
        start(0, 0)
        lax.fori_loop(0, count, body, 0)

    @pl.when(jnp.logical_and(wid == _NT - 1, count > 0))
    def _():
        start, body = make_parts(_NV_L, _STRIPE_L)
        start(0, 0)
        lax.fori_loop(0, count, body, 0)


def _merge_kernel(cv_ref, ci_ref, tail_ref, lp_ref, pen_ref, len_ref,
                  sc_ref, pv_ref, ix_ref):
    cv = cv_ref[...]
    ci = ci_ref[...]
    tail = tail_ref[...]
    lp = lp_ref[...]
    pen = pen_ref[...]
    done = len_ref[...] != 0
    B, K, W = cv.shape

    krow = lax.broadcasted_iota(jnp.int32, (B, K, W), 1)
    x_act = jnp.where(done, _NEG_INF, (lp + cv) / pen)
    i_act = jnp.where(done, _IMAX, krow * _V + ci)
    p_act = jnp.where(done, _NEG_INF, lp + cv)

    krowt = lax.broadcasted_iota(jnp.int32, (B, K, _TAILW), 1)
    colt = lax.broadcasted_iota(jnp.int32, (B, K, _TAILW), 2)
    x_tail = jnp.where(done, _NEG_INF, (lp + tail) / pen)
    i_tail = jnp.where(done, _IMAX, krowt * _V + _COV + colt)
    p_tail = jnp.where(done, _NEG_INF, lp + tail)

    krow1 = lax.broadcasted_iota(jnp.int32, (B, K, 1), 1)
    x_eos = jnp.where(done, lp / pen, _NEG_INF)
    i_eos = jnp.where(done, krow1 * _V + _EOS, _IMAX)
    p_eos = jnp.where(done, lp, _NEG_INF)

    X = jnp.concatenate([x_act, x_tail, x_eos], axis=2)
    I = jnp.concatenate([i_act, i_tail, i_eos], axis=2)
    P = jnp.concatenate([p_act, p_tail, p_eos], axis=2)

    scs, pvs, ixs = [], [], []
    for _ in range(5):
        m = jnp.max(jnp.max(X, axis=2), axis=1)
        mb = m[:, None, None]
        csel = jnp.min(jnp.min(jnp.where(X == mb, I, _IMAX), axis=2), axis=1)
        cb = csel[:, None, None]
        pv = jnp.max(jnp.max(jnp.where(I == cb, P, _NEG_INF), axis=2), axis=1)
        scs.append(m)
        pvs.append(pv)
        ixs.append(csel)
        X = jnp.where(I == cb, _NEG_INF, X)

    sc_ref[...] = jnp.stack(scs, axis=1)
    pv_ref[...] = jnp.stack(pvs, axis=1)
    ix_ref[...] = jnp.stack(ixs, axis=1)


def kernel(probs, log_probs, lengths, i, k):
    B, K, V = probs.shape

    len_pad = jnp.concatenate(
        [lengths.reshape(B * K), jnp.zeros(16, jnp.int32)])

    mesh = plsc.VectorSubcoreMesh(core_axis_name="c", subcore_axis_name="s")
    scan = pl.kernel(
        _sc_scan,
        out_type=(
            jax.ShapeDtypeStruct((_B * _K * _NT * _CW,), jnp.float32),
            jax.ShapeDtypeStruct((_B * _K * _NT * _CW,), jnp.int32),
        ),
        mesh=mesh,
        scratch_types=[
            pltpu.VMEM((336,), jnp.int32),
            pltpu.SMEM((88,), jnp.int32),
            pltpu.VMEM((2, _K, _STRIPE), jnp.float32),
            pltpu.VMEM((_CW,), jnp.float32),
            pltpu.VMEM((_CW,), jnp.int32),
            pltpu.SemaphoreType.DMA((2,)),
        ],
    )
    cv, ci = scan(probs, len_pad)

    eff = jnp.where(lengths == 0, i + 1, lengths).astype(jnp.float32)
    pen = jnp.power((5.0 + eff) / 6.0, 0.8)

    W = _NT * _CW
    tail = lax.slice(probs, (0, 0, _COV), (B, K, V))
    full = lambda shp: pl.BlockSpec(shp, lambda: (0,) * len(shp))
    sc, pv, ix = pl.pallas_call(
        _merge_kernel,
        in_specs=[
            full((B, K, W)), full((B, K, W)), full((B, K, _TAILW)),
            full((B, K, 1)), full((B, K, 1)), full((B, K, 1)),
        ],
        out_specs=(full((B, 5)), full((B, 5)), full((B, 5))),
        out_shape=(
            jax.ShapeDtypeStruct((B, 5), jnp.float32),
            jax.ShapeDtypeStruct((B, 5), jnp.float32),
            jax.ShapeDtypeStruct((B, 5), jnp.int32),
        ),
    )(cv.reshape(B, K, W), ci.reshape(B, K, W), tail,
      log_probs.reshape(B, K, 1), pen.reshape(B, K, 1),
      lengths.reshape(B, K, 1))

    best_idx = ix + jnp.asarray(k - K, jnp.int32)
    best_beams = best_idx // V
    best_tokens = best_idx % V
    return sc, pv, best_beams, best_tokens
